# baseline (speedup 1.0000x reference)
import jax
import jax.numpy as jnp
from jax import lax
from jax.experimental import pallas as pl
from jax.experimental.pallas import tpu as pltpu
from jax.experimental.pallas import tpu_sc as plsc
import functools

B = 4096
S = 200
D = 32

ROWS = B * S


TC_BLK = 1600


def _tc_body(idx_smem, table_any, pos_v, out_v, gbuf, sem):
    def enq(r, _):
        t = idx_smem[0, 0, r]
        pltpu.make_async_copy(table_any.at[pl.ds(t, 1)],
                              gbuf.at[pl.ds(r, 1)], sem).start()
        return 0

    lax.fori_loop(0, TC_BLK, enq, 0)
    pltpu.make_async_copy(table_any.at[pl.ds(0, TC_BLK)], gbuf, sem).wait()
    out_v[...] = (gbuf[...].reshape(TC_BLK // S, S, D)
                  + pos_v[...][None]).reshape(TC_BLK, D)


def _tc_gather(x_flat, token_table, pos_table):
    n_rows = x_flat.shape[0]
    grid = (n_rows // TC_BLK,)
    x_3d = x_flat.reshape(n_rows // TC_BLK, 1, TC_BLK)
    return pl.pallas_call(
        _tc_body,
        grid=grid,
        in_specs=[
            pl.BlockSpec((1, 1, TC_BLK), lambda i: (i, 0, 0),
                         memory_space=pltpu.SMEM),
            pl.BlockSpec(memory_space=pl.ANY),
            pl.BlockSpec((S, D), lambda i: (0, 0)),
        ],
        out_specs=pl.BlockSpec((TC_BLK, D), lambda i: (i, 0)),
        out_shape=jax.ShapeDtypeStruct((n_rows, D), jnp.float32),
        scratch_shapes=[
            pltpu.VMEM((TC_BLK, D), jnp.float32),
            pltpu.SemaphoreType.DMA,
        ],
    )(x_3d, token_table, pos_table)


def kernel(x, token_table, pos_table):
    x_flat = x.reshape(ROWS).astype(jnp.int32)
    out = _tc_gather(x_flat, token_table, pos_table)
    return out.reshape(B, S, D)


# speedup vs baseline: 3.1873x; 3.1873x over previous
import jax
import jax.numpy as jnp
from jax import lax
from jax.experimental import pallas as pl
from jax.experimental.pallas import tpu as pltpu
from jax.experimental.pallas import tpu_sc as plsc
import functools

B = 4096
S = 200
D = 32

NC = 2
NS = 16
NW = NC * NS

ROWS = B * S
CHUNK = 800
SUB = 80
NSUB = CHUNK // SUB

SC_CHUNKS = 28
SC_ROWS = NW * SC_CHUNKS * CHUNK
TC_BLK = 1600
TC_ROWS = ROWS - SC_ROWS

_mesh = plsc.VectorSubcoreMesh(core_axis_name="c", subcore_axis_name="s")


@functools.lru_cache(maxsize=None)
def _make_sc(nchunk, out_rows):
    @functools.partial(
        pl.kernel,
        mesh=_mesh,
        compiler_params=pltpu.CompilerParams(use_tc_tiling_on_sc=False),
        out_type=jax.ShapeDtypeStruct((out_rows, D), jnp.float32),
        scratch_types=[
            pltpu.VMEM((NSUB, SUB), jnp.int32),
            pltpu.VMEM((NSUB, SUB), jnp.int32),
            pltpu.VMEM((CHUNK, D), jnp.float32),
            pltpu.VMEM((CHUNK, D), jnp.float32),
            pltpu.VMEM((S, D), jnp.float32),
            pltpu.SemaphoreType.DMA,
            pltpu.SemaphoreType.DMA,
            pltpu.SemaphoreType.DMA,
            pltpu.SemaphoreType.DMA,
        ],
    )
    def _embed_sc(x_hbm, tok_hbm, pos_hbm, out_hbm,
                  idx0, idx1, rows0, rows1, pos_v, g0, g1, o0, o1):
        wid = lax.axis_index("s") * NC + lax.axis_index("c")
        idx = (idx0, idx1)
        rows = (rows0, rows1)
        gs = (g0, g1)
        os_ = (o0, o1)

        pltpu.sync_copy(pos_hbm, pos_v)

        def prefetch(c, b):
            pltpu.sync_copy(x_hbm.at[wid, c], idx[b])
            for j in range(NSUB):
                pltpu.async_copy(tok_hbm.at[idx[b].at[j]],
                                 rows[b].at[pl.ds(j * SUB, SUB)], gs[b])

        prefetch(0, 0)

        def outer(c2, _):
            for b in range(2):
                c = c2 * 2 + b
                nb = 1 - b

                @pl.when(c < nchunk - 1)
                def _():
                    @pl.when(c >= 1)
                    def _():
                        pltpu.make_async_copy(
                            rows[nb], out_hbm.at[pl.ds(0, CHUNK)],
                            os_[nb]).wait()
                    prefetch(c + 1, nb)

                pltpu.make_async_copy(
                    out_hbm.at[pl.ds(0, CHUNK)], rows[b], gs[b]).wait()

                def row_body(i2, _):
                    for k in range(8):
                        i = i2 * 8 + k
                        for h in range(D // 16):
                            sl = pl.ds(h * 16, 16)
                            pv = pos_v[i, sl]
                            for q in range(CHUNK // S):
                                r = q * S + i
                                rows[b][r, sl] = rows[b][r, sl] + pv
                    return 0

                lax.fori_loop(0, S // 8, row_body, 0)
                pltpu.async_copy(
                    rows[b],
                    out_hbm.at[pl.ds((wid * nchunk + c) * CHUNK, CHUNK)],
                    os_[b])
            return 0

        lax.fori_loop(0, nchunk // 2, outer, 0)
        pltpu.make_async_copy(rows[0], out_hbm.at[pl.ds(0, CHUNK)],
                              os_[0]).wait()
        pltpu.make_async_copy(rows[1], out_hbm.at[pl.ds(0, CHUNK)],
                              os_[1]).wait()

    return _embed_sc


def _tc_body(idx_smem, table_any, pos_v, out_v, gbuf, sem):
    def enq(r, _):
        t = idx_smem[0, 0, r]
        pltpu.make_async_copy(table_any.at[pl.ds(t, 1)],
                              gbuf.at[pl.ds(r, 1)], sem).start()
        return 0

    lax.fori_loop(0, TC_BLK, enq, 0)
    pltpu.make_async_copy(table_any.at[pl.ds(0, TC_BLK)], gbuf, sem).wait()
    out_v[...] = (gbuf[...].reshape(TC_BLK // S, S, D)
                  + pos_v[...][None]).reshape(TC_BLK, D)


def _tc_gather(x_flat, token_table, pos_table):
    n_rows = x_flat.shape[0]
    grid = (n_rows // TC_BLK,)
    x_3d = x_flat.reshape(n_rows // TC_BLK, 1, TC_BLK)
    return pl.pallas_call(
        _tc_body,
        grid=grid,
        in_specs=[
            pl.BlockSpec((1, 1, TC_BLK), lambda i: (i, 0, 0),
                         memory_space=pltpu.SMEM),
            pl.BlockSpec(memory_space=pl.ANY),
            pl.BlockSpec((S, D), lambda i: (0, 0)),
        ],
        out_specs=pl.BlockSpec((TC_BLK, D), lambda i: (i, 0)),
        out_shape=jax.ShapeDtypeStruct((n_rows, D), jnp.float32),
        scratch_shapes=[
            pltpu.VMEM((TC_BLK, D), jnp.float32),
            pltpu.SemaphoreType.DMA,
        ],
    )(x_3d, token_table, pos_table)


def kernel(x, token_table, pos_table):
    x_flat = x.reshape(ROWS).astype(jnp.int32)
    x_sc = x_flat[:SC_ROWS].reshape(NW, SC_CHUNKS, NSUB, SUB)
    out_sc = _make_sc(SC_CHUNKS, ROWS)(x_sc, token_table, pos_table)
    out_tc = _tc_gather(x_flat[SC_ROWS:], token_table, pos_table)
    out = lax.dynamic_update_slice(out_sc, out_tc, (SC_ROWS, 0))
    return out.reshape(B, S, D)


# speedup vs baseline: 5.5231x; 1.7328x over previous
import jax
import jax.numpy as jnp
from jax import lax
from jax.experimental import pallas as pl
from jax.experimental.pallas import tpu as pltpu
from jax.experimental.pallas import tpu_sc as plsc
import functools

B = 4096
S = 200
D = 32

NC = 2
NS = 16
NW = NC * NS

ROWS = B * S
ROWS_PER_W = ROWS // NW
CHUNK = 800
NCHUNK = ROWS_PER_W // CHUNK
SUB = 40
NSUB = CHUNK // SUB

_mesh = plsc.VectorSubcoreMesh(core_axis_name="c", subcore_axis_name="s")


@functools.partial(
    pl.kernel,
    mesh=_mesh,
    compiler_params=pltpu.CompilerParams(use_tc_tiling_on_sc=False),
    out_type=jax.ShapeDtypeStruct((ROWS, D), jnp.float32),
    scratch_types=[
        pltpu.VMEM((NSUB, SUB), jnp.int32),
        pltpu.VMEM((NSUB, SUB), jnp.int32),
        pltpu.VMEM((CHUNK, D), jnp.float32),
        pltpu.VMEM((CHUNK, D), jnp.float32),
        pltpu.VMEM((S, D), jnp.float32),
        pltpu.SemaphoreType.DMA,
        pltpu.SemaphoreType.DMA,
        pltpu.SemaphoreType.DMA,
        pltpu.SemaphoreType.DMA,
    ],
)
def _embed_sc(x_hbm, tok_hbm, pos_hbm, out_hbm,
              idx0, idx1, rows0, rows1, pos_v, g0, g1, o0, o1):
    wid = lax.axis_index("s") * NC + lax.axis_index("c")
    idx = (idx0, idx1)
    rows = (rows0, rows1)
    gs = (g0, g1)
    os_ = (o0, o1)

    pltpu.sync_copy(pos_hbm, pos_v)

    def prefetch(c, b):
        pltpu.sync_copy(x_hbm.at[wid, c], idx[b])
        for j in range(NSUB):
            pltpu.async_copy(tok_hbm.at[idx[b].at[j]],
                             rows[b].at[pl.ds(j * SUB, SUB)], gs[b])

    prefetch(0, 0)

    def outer(c2, _):
        for b in range(2):
            c = c2 * 2 + b
            nb = 1 - b

            @pl.when(c < NCHUNK - 1)
            def _():
                @pl.when(c >= 1)
                def _():
                    pltpu.make_async_copy(
                        rows[nb], out_hbm.at[pl.ds(0, CHUNK)], os_[nb]).wait()
                prefetch(c + 1, nb)

            pltpu.make_async_copy(
                out_hbm.at[pl.ds(0, CHUNK)], rows[b], gs[b]).wait()

            def row_body(i2, _):
                for k in range(8):
                    i = i2 * 8 + k
                    for h in range(D // 16):
                        sl = pl.ds(h * 16, 16)
                        pv = pos_v[i, sl]
                        for q in range(CHUNK // S):
                            r = q * S + i
                            rows[b][r, sl] = rows[b][r, sl] + pv
                return 0

            lax.fori_loop(0, S // 8, row_body, 0)
            pltpu.async_copy(
                rows[b],
                out_hbm.at[pl.ds((wid * NCHUNK + c) * CHUNK, CHUNK)], os_[b])
        return 0

    lax.fori_loop(0, NCHUNK // 2, outer, 0)
    pltpu.make_async_copy(rows[0], out_hbm.at[pl.ds(0, CHUNK)], os_[0]).wait()
    pltpu.make_async_copy(rows[1], out_hbm.at[pl.ds(0, CHUNK)], os_[1]).wait()


def kernel(x, token_table, pos_table):
    x_r = x.reshape(NW, NCHUNK, NSUB, SUB).astype(jnp.int32)
    out = _embed_sc(x_r, token_table, pos_table)
    return out.reshape(B, S, D)
